# baseline (speedup 1.0000x reference)
import jax
import jax.numpy as jnp
from jax import lax
from jax.experimental import pallas as pl
from jax.experimental.pallas import tpu as pltpu
from jax.experimental.pallas import tpu_sc as plsc

NTOK = 1000
B, L = 1024, 50
ROWS = B * L
NC, NS = 2, 16
NW = NC * NS
RPW = ROWS // NW
CHUNK = 64
NCHUNK = RPW // CHUNK
CBUF = CHUNK * NTOK
LANES = 16


def _body(x_hbm, out_hbm, idx_v, buf0, buf1, sem0, sem1):
    wid = lax.axis_index("s") * NC + lax.axis_index("c")
    base_row = wid * RPW

    pltpu.sync_copy(x_hbm.at[pl.ds(base_row, RPW)], idx_v)

    def _zero(i):
        z = jnp.zeros((LANES,), jnp.float32)
        buf0[pl.ds(i * LANES, LANES)] = z
        buf1[pl.ds(i * LANES, LANES)] = z

    pl.loop(0, CBUF // LANES)(_zero)

    bufs = (buf0, buf1)
    sems = (sem0, sem1)
    lane = lax.iota(jnp.int32, LANES)
    ones = jnp.ones((LANES,), jnp.float32)
    zeros = jnp.zeros((LANES,), jnp.float32)

    def scatter(buf, chunk, value):
        for k in range(CHUNK // LANES):
            vals = idx_v[pl.ds(chunk * CHUNK + k * LANES, LANES)]
            ok = (vals >= 0) & (vals < NTOK)
            pos = (k * LANES + lane) * NTOK + jnp.where(ok, vals, 0)
            plsc.store_scatter(buf, [pos], value, mask=ok)

    copies = [None, None]
    for j in range(NCHUNK):
        s = j % 2
        if copies[s] is not None:
            copies[s].wait()
            scatter(bufs[s], j - 2, zeros)
        scatter(bufs[s], j, ones)
        copies[s] = pltpu.async_copy(
            bufs[s],
            out_hbm.at[pl.ds((base_row + j * CHUNK) * NTOK, CBUF)],
            sems[s],
        )
    copies[(NCHUNK - 1) % 2].wait()
    copies[NCHUNK % 2].wait()


@jax.jit
def kernel(x):
    mesh = plsc.VectorSubcoreMesh(
        core_axis_name="c", subcore_axis_name="s",
        num_cores=NC, num_subcores=NS,
    )
    run = pl.kernel(
        _body,
        out_type=jax.ShapeDtypeStruct((ROWS * NTOK,), jnp.float32),
        mesh=mesh,
        scratch_types=[
            pltpu.VMEM((RPW,), jnp.int32),
            pltpu.VMEM((CBUF,), jnp.float32),
            pltpu.VMEM((CBUF,), jnp.float32),
            pltpu.SemaphoreType.DMA,
            pltpu.SemaphoreType.DMA,
        ],
        compiler_params=pltpu.CompilerParams(needs_layout_passes=False),
    )
    flat = run(x.reshape(ROWS).astype(jnp.int32))
    return flat.reshape(B, L, NTOK)
